# baseline (speedup 1.0000x reference)
import functools

import jax
import jax.numpy as jnp
from jax.experimental import pallas as pl

N = 10000
E = 160000
D = 256
H = 8
C = 128
NC = 40


def _matmul_kernel(x_ref, w_ref, b_ref, o_ref):
    o_ref[...] = jnp.dot(x_ref[...], w_ref[...],
                         preferred_element_type=jnp.float32) + b_ref[...]


def _project(x, w, b, block_rows=1000):
    n, d = x.shape
    k = w.shape[1]
    return pl.pallas_call(
        _matmul_kernel,
        grid=(n // block_rows,),
        in_specs=[
            pl.BlockSpec((block_rows, d), lambda i: (i, 0)),
            pl.BlockSpec((d, k), lambda i: (0, 0)),
            pl.BlockSpec((k,), lambda i: (0,)),
        ],
        out_specs=pl.BlockSpec((block_rows, k), lambda i: (i, 0)),
        out_shape=jax.ShapeDtypeStruct((n, k), jnp.float32),
    )(x, w, b)


def _edge_phase(xl, xr, src, dst, att, heads, out_ch):
    xj = xl[src].reshape(-1, heads, out_ch)
    xi = xr[dst].reshape(-1, heads, out_ch)
    e = jax.nn.leaky_relu(xi + xj, negative_slope=0.2)
    logits = jnp.sum(e * att[None, :, :], axis=-1)
    m = jax.ops.segment_max(logits, dst, num_segments=N)
    m = jnp.where(jnp.isfinite(m), m, 0.0)
    ex = jnp.exp(logits - m[dst])
    denom = jax.ops.segment_sum(ex, dst, num_segments=N)
    alpha = ex / (denom[dst] + 1e-16)
    out = jax.ops.segment_sum(xj * alpha[:, :, None], dst, num_segments=N)
    return out.reshape(N, heads * out_ch)


def kernel(x, edge_index, W_l1, b_l1, W_r1, b_r1, att1, bias1,
           W_l2, b_l2, W_r2, b_r2, att2, bias2):
    loop = jnp.arange(N, dtype=edge_index.dtype)
    src = jnp.concatenate([edge_index[0], loop])
    dst = jnp.concatenate([edge_index[1], loop])

    xl = _project(x, W_l1, b_l1)
    xr = _project(x, W_r1, b_r1)
    h = _edge_phase(xl, xr, src, dst, att1, H, C) + bias1
    h = jax.nn.relu(h)

    xl2 = _project(h, W_l2, b_l2)
    xr2 = _project(h, W_r2, b_r2)
    out = _edge_phase(xl2, xr2, src, dst, att2, 1, NC) + bias2
    return out


# speedup vs baseline: 4.4114x; 4.4114x over previous
import functools

import jax
import jax.numpy as jnp
from jax import lax
from jax.experimental import pallas as pl
from jax.experimental.pallas import tpu as pltpu
from jax.experimental.pallas import tpu_sc as plsc

N = 10000
E = 160000
D = 256
H = 8
C = 128
NC = 40

NP = 10240
EP = 172032
CH = 128
NCHUNK = EP // CH
CPS = NCHUNK // 16
NPH = NP // 2
NT = 5136
NTR = 48
RPT = NPH // 16
ZCH = 16
NC2 = 64
L = 16
ROWB = 512
NB = NP // ROWB

_SC_MESH = dict(core_axis_name="c", subcore_axis_name="s",
                num_cores=2, num_subcores=16)
_SC_PARAMS = pltpu.CompilerParams(needs_layout_passes=False)


def _proj1_body(x_ref, wl_ref, wr_ref, bl_ref, br_ref, xl_ref, xr_ref):
    x = x_ref[...]
    xl_ref[...] = jnp.dot(x, wl_ref[...],
                          preferred_element_type=jnp.float32) + bl_ref[...]
    xr_ref[...] = jnp.dot(x, wr_ref[...],
                          preferred_element_type=jnp.float32) + br_ref[...]


def _proj1(xp, W_l1, b_l1, W_r1, b_r1):
    return pl.pallas_call(
        _proj1_body,
        grid=(H, NB),
        in_specs=[
            pl.BlockSpec((ROWB, D), lambda h, i: (i, 0)),
            pl.BlockSpec((D, C), lambda h, i: (0, h)),
            pl.BlockSpec((D, C), lambda h, i: (0, h)),
            pl.BlockSpec((C,), lambda h, i: (h,)),
            pl.BlockSpec((C,), lambda h, i: (h,)),
        ],
        out_specs=[
            pl.BlockSpec((ROWB, C), lambda h, i: (h * NB + i, 0)),
            pl.BlockSpec((ROWB, C), lambda h, i: (h * NB + i, 0)),
        ],
        out_shape=[
            jax.ShapeDtypeStruct((H * NP, C), jnp.float32),
            jax.ShapeDtypeStruct((H * NP, C), jnp.float32),
        ],
    )(xp, W_l1, W_r1, b_l1, b_r1)


@functools.partial(
    pl.kernel,
    out_type=[
        jax.ShapeDtypeStruct((H * NP, C), jnp.float32),
        jax.ShapeDtypeStruct((H * NP // 128, C), jnp.float32),
    ],
    mesh=plsc.VectorSubcoreMesh(**_SC_MESH),
    scratch_types=[
        pltpu.VMEM((CH,), jnp.int32),
        pltpu.VMEM((CH,), jnp.int32),
        pltpu.VMEM((CH,), jnp.int32),
        pltpu.VMEM((CH,), jnp.int32),
        pltpu.VMEM((CH,), jnp.int32),
        pltpu.VMEM((CH, C), jnp.float32),
        pltpu.VMEM((CH, C), jnp.float32),
        pltpu.VMEM((CH, L), jnp.float32),
        pltpu.VMEM((CH,), jnp.float32),
        pltpu.VMEM((CPS, CH), jnp.float32),
        pltpu.VMEM((C,), jnp.float32),
        pltpu.VMEM((ZCH, C), jnp.float32),
        pltpu.VMEM((NTR, C), jnp.float32),
        pltpu.VMEM((NTR,), jnp.int32),
        pltpu.VMEM_SHARED((NT, C), jnp.float32),
        pltpu.VMEM_SHARED((NTR, C), jnp.float32),
    ],
    compiler_params=_SC_PARAMS,
)
def _sc_layer1(xl_hbm, xr_hbm, src_hbm, dst_hbm, att_hbm, outn_hbm, outd_hbm,
               srcv, dstv, gl, gr, si, bufL, bufR, accb, exb, exall, attb,
               zb, denv, idr, tableN, tableD):
    c = lax.axis_index("c")
    s = lax.axis_index("s")
    iota = lax.iota(jnp.int32, L)
    zeros = jnp.zeros((L,), jnp.float32)

    @pl.loop(0, ZCH)
    def _zrow(r):
        for k in range(C // L):
            zb[r, pl.ds(L * k, L)] = zeros

    for k in range(NTR // L):
        idr[pl.ds(L * k, L)] = iota + L * k

    @pl.loop(0, 8)
    def _round(r):
        hh = r >> 1
        half = r & 1
        h = c * 4 + hh
        hNP = h * NP
        lo = half * NPH
        pltpu.sync_copy(att_hbm.at[h], attb)
        attr = [attb[pl.ds(L * v, L)] for v in range(C // L)]

        @pl.loop(0, RPT // 8)
        def _ztab(k):
            pltpu.sync_copy(zb.at[pl.ds(0, 8)],
                            tableN.at[pl.ds(s * RPT + k * 8, 8)])

        @pl.when(s < 2)
        def _ztabdum():
            pltpu.sync_copy(zb.at[pl.ds(0, 8)],
                            tableN.at[pl.ds(NPH + s * 8, 8)])

        @pl.when(s < 6)
        def _ztabd():
            pltpu.sync_copy(zb.at[pl.ds(0, 8)], tableD.at[pl.ds(s * 8, 8)])

        @pl.loop(0, NTR)
        def _zden(i):
            for k in range(C // L):
                denv[i, pl.ds(L * k, L)] = zeros
        plsc.subcore_barrier()

        @pl.loop(0, CPS)
        def _chunk(ci):
            base = s * (CPS * CH) + ci * CH
            pltpu.sync_copy(src_hbm.at[pl.ds(base, CH)], srcv)
            pltpu.sync_copy(dst_hbm.at[pl.ds(base, CH)], dstv)
            for k in range(CH // L):
                sl = srcv[pl.ds(L * k, L)]
                dl = dstv[pl.ds(L * k, L)]
                loc = dl - lo
                loc = jnp.where((loc >= 0) & (loc < NPH), loc,
                                NPH + (dl & 15))
                gl[pl.ds(L * k, L)] = sl + hNP
                gr[pl.ds(L * k, L)] = dl + hNP
                si[pl.ds(L * k, L)] = loc
            pltpu.sync_copy(xl_hbm.at[gl], bufL)

            @pl.when(half == 0)
            def _compute_logits():
                pltpu.sync_copy(xr_hbm.at[gr], bufR)

                @pl.loop(0, CH)
                def _edge(j):
                    acc = zeros
                    for v in range(C // L):
                        z = (bufL[j, pl.ds(L * v, L)]
                             + bufR[j, pl.ds(L * v, L)])
                        lz = jnp.maximum(z, 0.2 * z)
                        acc = acc + attr[v] * lz
                    accb[j, :] = acc

                for g in range(CH // L):
                    rows = g * L + iota
                    tot = zeros
                    for k in range(L):
                        tot = tot + plsc.load_gather(
                            accb, [rows, jnp.full((L,), k, jnp.int32)])
                    ex16 = jnp.exp(tot)
                    exb[pl.ds(g * L, L)] = ex16
                    exall[ci, pl.ds(g * L, L)] = ex16

            @pl.when(half == 1)
            def _reuse_logits():
                for g in range(CH // L):
                    exb[pl.ds(g * L, L)] = exall[ci, pl.ds(g * L, L)]

            for g in range(CH // L):
                sg = si[pl.ds(g * L, L)]
                plsc.addupdate_scatter(denv, [sg >> 7, sg & 127],
                                       exb[pl.ds(g * L, L)])

            @pl.loop(0, CH)
            def _scale(j):
                exs = plsc.load_gather(exb, [jnp.full((L,), j, jnp.int32)])
                for v in range(C // L):
                    bufL[j, pl.ds(L * v, L)] = bufL[j, pl.ds(L * v, L)] * exs

            pltpu.sync_copy(bufL, tableN.at[si], add=True)

        pltpu.sync_copy(denv, tableD.at[idr], add=True)
        plsc.subcore_barrier()
        pltpu.sync_copy(tableN.at[pl.ds(s * RPT, RPT)],
                        outn_hbm.at[pl.ds(hNP + lo + s * RPT, RPT)])

        @pl.when(s < 5)
        def _draind():
            pltpu.sync_copy(
                tableD.at[pl.ds(s * 8, 8)],
                outd_hbm.at[pl.ds(h * (NP // 128) + half * 40 + s * 8, 8)])


def _mid_body(o1n_ref, o1d_ref, wl_ref, wr_ref, bl_ref, br_ref, b1_ref,
              xl2_ref, xr2_ref):
    accl = jnp.zeros((ROWB, C), jnp.float32)
    accr = jnp.zeros((ROWB, C), jnp.float32)
    for h in range(H):
        den = o1d_ref[h, 0, 0][:, None] + 1e-16
        h1 = jnp.maximum(o1n_ref[h] / den + b1_ref[pl.ds(h * C, C)][None, :],
                         0.0)
        accl += jnp.dot(h1, wl_ref[pl.ds(h * C, C), :],
                        preferred_element_type=jnp.float32)
        accr += jnp.dot(h1, wr_ref[pl.ds(h * C, C), :],
                        preferred_element_type=jnp.float32)
    xl2_ref[...] = accl + bl_ref[...]
    xr2_ref[...] = accr + br_ref[...]


def _mid(out1n, out1d, W_l2p, b_l2p, W_r2p, b_r2p, bias1):
    return pl.pallas_call(
        _mid_body,
        grid=(NB,),
        in_specs=[
            pl.BlockSpec((H, ROWB, C), lambda i: (0, i, 0)),
            pl.BlockSpec((H, 1, 1, ROWB), lambda i: (0, i, 0, 0)),
            pl.BlockSpec((H * C, C), lambda i: (0, 0)),
            pl.BlockSpec((H * C, C), lambda i: (0, 0)),
            pl.BlockSpec((C,), lambda i: (0,)),
            pl.BlockSpec((C,), lambda i: (0,)),
            pl.BlockSpec((H * C,), lambda i: (0,)),
        ],
        out_specs=[
            pl.BlockSpec((ROWB, C), lambda i: (i, 0)),
            pl.BlockSpec((ROWB, C), lambda i: (i, 0)),
        ],
        out_shape=[
            jax.ShapeDtypeStruct((NP, C), jnp.float32),
            jax.ShapeDtypeStruct((NP, C), jnp.float32),
        ],
    )(out1n.reshape(H, NP, C), out1d.reshape(H, NB, 1, ROWB),
      W_l2p, W_r2p, b_l2p, b_r2p, bias1)


@functools.partial(
    pl.kernel,
    out_type=jax.ShapeDtypeStruct((NP, C), jnp.float32),
    mesh=plsc.VectorSubcoreMesh(**_SC_MESH),
    scratch_types=[
        pltpu.VMEM((CH,), jnp.int32),
        pltpu.VMEM((CH,), jnp.int32),
        pltpu.VMEM((CH,), jnp.int32),
        pltpu.VMEM((CH,), jnp.int32),
        pltpu.VMEM((CH,), jnp.int32),
        pltpu.VMEM((CH, C), jnp.float32),
        pltpu.VMEM((CH, C), jnp.float32),
        pltpu.VMEM((CH, L), jnp.float32),
        pltpu.VMEM((CH,), jnp.float32),
        pltpu.VMEM((C,), jnp.float32),
        pltpu.VMEM((ZCH, C), jnp.float32),
        pltpu.VMEM_SHARED((NT, C), jnp.float32),
    ],
    compiler_params=_SC_PARAMS,
)
def _sc_layer2(xl_hbm, xr_hbm, src_hbm, dst_hbm, att_hbm, outn_hbm,
               srcv, dstv, gl, gr, si, bufL, bufR, accb, exb, attb,
               zb, tableN):
    c = lax.axis_index("c")
    s = lax.axis_index("s")
    iota = lax.iota(jnp.int32, L)
    zeros = jnp.zeros((L,), jnp.float32)
    lo = c * NPH

    pltpu.sync_copy(att_hbm, attb)
    attr = [attb[pl.ds(L * v, L)] for v in range(C // L)]

    @pl.loop(0, ZCH)
    def _zrow(r):
        for k in range(C // L):
            zb[r, pl.ds(L * k, L)] = zeros

    @pl.loop(0, RPT // 8)
    def _ztab(k):
        pltpu.sync_copy(zb.at[pl.ds(0, 8)],
                        tableN.at[pl.ds(s * RPT + k * 8, 8)])

    @pl.when(s < 2)
    def _ztabdum():
        pltpu.sync_copy(zb.at[pl.ds(0, 8)],
                        tableN.at[pl.ds(NPH + s * 8, 8)])
    plsc.subcore_barrier()

    @pl.loop(0, CPS)
    def _chunk(ci):
        base = s * (CPS * CH) + ci * CH
        pltpu.sync_copy(src_hbm.at[pl.ds(base, CH)], srcv)
        pltpu.sync_copy(dst_hbm.at[pl.ds(base, CH)], dstv)
        for k in range(CH // L):
            dl = dstv[pl.ds(L * k, L)]
            loc = dl - lo
            loc = jnp.where((loc >= 0) & (loc < NPH), loc, NPH + (dl & 15))
            gl[pl.ds(L * k, L)] = srcv[pl.ds(L * k, L)]
            gr[pl.ds(L * k, L)] = dl
            si[pl.ds(L * k, L)] = loc
        pltpu.sync_copy(xl_hbm.at[gl], bufL)
        pltpu.sync_copy(xr_hbm.at[gr], bufR)

        @pl.loop(0, CH)
        def _edge(j):
            acc = zeros
            for v in range(C // L):
                z = bufL[j, pl.ds(L * v, L)] + bufR[j, pl.ds(L * v, L)]
                lz = jnp.maximum(z, 0.2 * z)
                acc = acc + attr[v] * lz
            accb[j, :] = acc

        for g in range(CH // L):
            rows = g * L + iota
            tot = zeros
            for k in range(L):
                tot = tot + plsc.load_gather(
                    accb, [rows, jnp.full((L,), k, jnp.int32)])
            exb[pl.ds(g * L, L)] = jnp.exp(tot)

        @pl.loop(0, CH)
        def _scale(j):
            exs = plsc.load_gather(exb, [jnp.full((L,), j, jnp.int32)])
            for v in range(NC2 // L):
                bufL[j, pl.ds(L * v, L)] = bufL[j, pl.ds(L * v, L)] * exs
            bufL[j, pl.ds(NC2, L)] = jnp.where(iota == 0, exs, 0.0)

        pltpu.sync_copy(bufL, tableN.at[si], add=True)

    plsc.subcore_barrier()
    pltpu.sync_copy(tableN.at[pl.ds(s * RPT, RPT)],
                    outn_hbm.at[pl.ds(lo + s * RPT, RPT)])


def _fin_body(p2n_ref, b2_ref, o_ref):
    den = p2n_ref[:, NC2:NC2 + 1] + 1e-16
    o_ref[...] = p2n_ref[:, :NC] / den + b2_ref[...][None, :]


def _fin(p2n, bias2):
    return pl.pallas_call(
        _fin_body,
        grid=(NB,),
        in_specs=[
            pl.BlockSpec((ROWB, C), lambda i: (i, 0)),
            pl.BlockSpec((NC,), lambda i: (0,)),
        ],
        out_specs=pl.BlockSpec((ROWB, NC), lambda i: (i, 0)),
        out_shape=jax.ShapeDtypeStruct((NP, NC), jnp.float32),
    )(p2n, bias2)


def kernel(x, edge_index, W_l1, b_l1, W_r1, b_r1, att1, bias1,
           W_l2, b_l2, W_r2, b_r2, att2, bias2):
    xp = jnp.pad(x, ((0, NP - N), (0, 0)))
    loop = jnp.arange(N, dtype=jnp.int32)
    pad = jnp.full((EP - E - N,), N, jnp.int32)
    src = jnp.concatenate([edge_index[0], loop, pad])
    dst = jnp.concatenate([edge_index[1], loop, pad])

    xl1, xr1 = _proj1(xp, W_l1, b_l1, W_r1, b_r1)
    out1n, out1d = _sc_layer1(xl1, xr1, src, dst, att1)

    W_l2p = jnp.pad(W_l2, ((0, 0), (0, C - NC)))
    W_r2p = jnp.pad(W_r2, ((0, 0), (0, C - NC)))
    b_l2p = jnp.pad(b_l2, (0, C - NC))
    b_r2p = jnp.pad(b_r2, (0, C - NC))
    att2p = jnp.pad(att2[0], (0, C - NC))

    xl2, xr2 = _mid(out1n, out1d, W_l2p, b_l2p, W_r2p, b_r2p, bias1)
    p2n = _sc_layer2(xl2, xr2, src, dst, att2p)
    return _fin(p2n, bias2)[:N]


# speedup vs baseline: 5.3076x; 1.2032x over previous
import functools

import jax
import jax.numpy as jnp
from jax import lax
from jax.experimental import pallas as pl
from jax.experimental.pallas import tpu as pltpu
from jax.experimental.pallas import tpu_sc as plsc

N = 10000
E = 160000
D = 256
H = 8
C = 128
NC = 40

NP = 10240
EP = 172032
CH = 128
NCHUNK = EP // CH
CPS = NCHUNK // 16
NPH = NP // 2
NT = 5136
NTR = 48
RPT = NPH // 16
ZCH = 16
NC2 = 64
L = 16
ROWB = 512
NB = NP // ROWB

_SC_MESH = dict(core_axis_name="c", subcore_axis_name="s",
                num_cores=2, num_subcores=16)
_SC_PARAMS = pltpu.CompilerParams(needs_layout_passes=False)


def _proj1_body(x_ref, wl_ref, wr_ref, bl_ref, br_ref, xl_ref, xr_ref):
    x = x_ref[...]
    xl_ref[...] = jnp.dot(x, wl_ref[...],
                          preferred_element_type=jnp.float32) + bl_ref[...]
    xr_ref[...] = jnp.dot(x, wr_ref[...],
                          preferred_element_type=jnp.float32) + br_ref[...]


def _proj1(xp, W_l1, b_l1, W_r1, b_r1):
    return pl.pallas_call(
        _proj1_body,
        grid=(H, NB),
        in_specs=[
            pl.BlockSpec((ROWB, D), lambda h, i: (i, 0)),
            pl.BlockSpec((D, C), lambda h, i: (0, h)),
            pl.BlockSpec((D, C), lambda h, i: (0, h)),
            pl.BlockSpec((C,), lambda h, i: (h,)),
            pl.BlockSpec((C,), lambda h, i: (h,)),
        ],
        out_specs=[
            pl.BlockSpec((ROWB, C), lambda h, i: (h * NB + i, 0)),
            pl.BlockSpec((ROWB, C), lambda h, i: (h * NB + i, 0)),
        ],
        out_shape=[
            jax.ShapeDtypeStruct((H * NP, C), jnp.float32),
            jax.ShapeDtypeStruct((H * NP, C), jnp.float32),
        ],
    )(xp, W_l1, W_r1, b_l1, b_r1)


@functools.partial(
    pl.kernel,
    out_type=[
        jax.ShapeDtypeStruct((H * NP, C), jnp.float32),
        jax.ShapeDtypeStruct((H * NP // 128, C), jnp.float32),
    ],
    mesh=plsc.VectorSubcoreMesh(**_SC_MESH),
    scratch_types=[
        pltpu.VMEM((CH,), jnp.int32),
        pltpu.VMEM((CH,), jnp.int32),
        pltpu.VMEM((CH,), jnp.int32),
        pltpu.VMEM((CH,), jnp.int32),
        pltpu.VMEM((CH,), jnp.int32),
        pltpu.VMEM((CH, C), jnp.float32),
        pltpu.VMEM((CH, C), jnp.float32),
        pltpu.VMEM((CH, L), jnp.float32),
        pltpu.VMEM((CH,), jnp.float32),
        pltpu.VMEM((CPS, CH), jnp.float32),
        pltpu.VMEM((C,), jnp.float32),
        pltpu.VMEM((ZCH, C), jnp.float32),
        pltpu.VMEM((NTR, C), jnp.float32),
        pltpu.VMEM((NTR,), jnp.int32),
        pltpu.VMEM_SHARED((NT, C), jnp.float32),
        pltpu.VMEM_SHARED((NTR, C), jnp.float32),
        pltpu.SemaphoreType.DMA,
        pltpu.SemaphoreType.DMA,
    ],
    compiler_params=_SC_PARAMS,
)
def _sc_layer1(xl_hbm, xr_hbm, src_hbm, dst_hbm, att_hbm, outn_hbm, outd_hbm,
               srcv, dstv, gl, gr, si, bufL, bufR, accb, exb, exall, attb,
               zb, denv, idr, tableN, tableD, semA, semB):
    c = lax.axis_index("c")
    s = lax.axis_index("s")
    iota = lax.iota(jnp.int32, L)
    zeros = jnp.zeros((L,), jnp.float32)

    @pl.loop(0, ZCH)
    def _zrow(r):
        for k in range(C // L):
            zb[r, pl.ds(L * k, L)] = zeros

    for k in range(NTR // L):
        idr[pl.ds(L * k, L)] = iota + L * k

    @pl.loop(0, 8)
    def _round(r):
        hh = r >> 1
        half = r & 1
        h = c * 4 + hh
        hNP = h * NP
        lo = half * NPH
        pltpu.sync_copy(att_hbm.at[h], attb)
        attr = [attb[pl.ds(L * v, L)] for v in range(C // L)]

        @pl.loop(0, RPT // 8)
        def _ztab(k):
            pltpu.sync_copy(zb.at[pl.ds(0, 8)],
                            tableN.at[pl.ds(s * RPT + k * 8, 8)])

        @pl.when(s < 2)
        def _ztabdum():
            pltpu.sync_copy(zb.at[pl.ds(0, 8)],
                            tableN.at[pl.ds(NPH + s * 8, 8)])

        @pl.when(s < 6)
        def _ztabd():
            pltpu.sync_copy(zb.at[pl.ds(0, 8)], tableD.at[pl.ds(s * 8, 8)])

        @pl.loop(0, NTR)
        def _zden(i):
            for k in range(C // L):
                denv[i, pl.ds(L * k, L)] = zeros
        plsc.subcore_barrier()

        @pl.loop(0, CPS)
        def _chunk(ci):
            base = s * (CPS * CH) + ci * CH
            h1 = pltpu.async_copy(src_hbm.at[pl.ds(base, CH)], srcv, semA)
            h2 = pltpu.async_copy(dst_hbm.at[pl.ds(base, CH)], dstv, semB)
            h1.wait()
            h2.wait()
            for k in range(CH // L):
                sl = srcv[pl.ds(L * k, L)]
                dl = dstv[pl.ds(L * k, L)]
                loc = dl - lo
                loc = jnp.where((loc >= 0) & (loc < NPH), loc,
                                NPH + (dl & 15))
                gl[pl.ds(L * k, L)] = sl + hNP
                gr[pl.ds(L * k, L)] = dl + hNP
                si[pl.ds(L * k, L)] = loc
            g1 = pltpu.async_copy(xl_hbm.at[gl], bufL, semA)

            @pl.when(half == 0)
            def _gather_r():
                pltpu.async_copy(xr_hbm.at[gr], bufR, semB).wait()

            g1.wait()

            @pl.when(half == 0)
            def _compute_logits():
                @pl.loop(0, CH)
                def _edge(j):
                    acc = zeros
                    for v in range(C // L):
                        z = (bufL[j, pl.ds(L * v, L)]
                             + bufR[j, pl.ds(L * v, L)])
                        lz = jnp.maximum(z, 0.2 * z)
                        acc = acc + attr[v] * lz
                    accb[j, :] = acc

                for g in range(CH // L):
                    rows = g * L + iota
                    tot = zeros
                    for k in range(L):
                        tot = tot + plsc.load_gather(
                            accb, [rows, jnp.full((L,), k, jnp.int32)])
                    ex16 = jnp.exp(tot)
                    exb[pl.ds(g * L, L)] = ex16
                    exall[ci, pl.ds(g * L, L)] = ex16

            @pl.when(half == 1)
            def _reuse_logits():
                for g in range(CH // L):
                    exb[pl.ds(g * L, L)] = exall[ci, pl.ds(g * L, L)]

            for g in range(CH // L):
                sg = si[pl.ds(g * L, L)]
                plsc.addupdate_scatter(denv, [sg >> 7, sg & 127],
                                       exb[pl.ds(g * L, L)])

            @pl.loop(0, CH)
            def _scale(j):
                exs = plsc.load_gather(exb, [jnp.full((L,), j, jnp.int32)])
                for v in range(C // L):
                    bufL[j, pl.ds(L * v, L)] = bufL[j, pl.ds(L * v, L)] * exs

            pltpu.sync_copy(bufL, tableN.at[si], add=True)

        pltpu.sync_copy(denv, tableD.at[idr], add=True)
        plsc.subcore_barrier()
        pltpu.sync_copy(tableN.at[pl.ds(s * RPT, RPT)],
                        outn_hbm.at[pl.ds(hNP + lo + s * RPT, RPT)])

        @pl.when(s < 5)
        def _draind():
            pltpu.sync_copy(
                tableD.at[pl.ds(s * 8, 8)],
                outd_hbm.at[pl.ds(h * (NP // 128) + half * 40 + s * 8, 8)])


def _mid_body(o1n_ref, o1d_ref, wl_ref, wr_ref, bl_ref, br_ref, b1_ref,
              xl2_ref, xr2_ref):
    accl = jnp.zeros((ROWB, C), jnp.float32)
    accr = jnp.zeros((ROWB, C), jnp.float32)
    for h in range(H):
        den = o1d_ref[h, 0, 0][:, None] + 1e-16
        h1 = jnp.maximum(o1n_ref[h] / den + b1_ref[pl.ds(h * C, C)][None, :],
                         0.0)
        accl += jnp.dot(h1, wl_ref[pl.ds(h * C, C), :],
                        preferred_element_type=jnp.float32)
        accr += jnp.dot(h1, wr_ref[pl.ds(h * C, C), :],
                        preferred_element_type=jnp.float32)
    xl2_ref[...] = accl + bl_ref[...]
    xr2_ref[...] = accr + br_ref[...]


def _mid(out1n, out1d, W_l2p, b_l2p, W_r2p, b_r2p, bias1):
    return pl.pallas_call(
        _mid_body,
        grid=(NB,),
        in_specs=[
            pl.BlockSpec((H, ROWB, C), lambda i: (0, i, 0)),
            pl.BlockSpec((H, 1, 1, ROWB), lambda i: (0, i, 0, 0)),
            pl.BlockSpec((H * C, C), lambda i: (0, 0)),
            pl.BlockSpec((H * C, C), lambda i: (0, 0)),
            pl.BlockSpec((C,), lambda i: (0,)),
            pl.BlockSpec((C,), lambda i: (0,)),
            pl.BlockSpec((H * C,), lambda i: (0,)),
        ],
        out_specs=[
            pl.BlockSpec((ROWB, C), lambda i: (i, 0)),
            pl.BlockSpec((ROWB, C), lambda i: (i, 0)),
        ],
        out_shape=[
            jax.ShapeDtypeStruct((NP, C), jnp.float32),
            jax.ShapeDtypeStruct((NP, C), jnp.float32),
        ],
    )(out1n.reshape(H, NP, C), out1d.reshape(H, NB, 1, ROWB),
      W_l2p, W_r2p, b_l2p, b_r2p, bias1)


@functools.partial(
    pl.kernel,
    out_type=jax.ShapeDtypeStruct((NP, C), jnp.float32),
    mesh=plsc.VectorSubcoreMesh(**_SC_MESH),
    scratch_types=[
        pltpu.VMEM((CH,), jnp.int32),
        pltpu.VMEM((CH,), jnp.int32),
        pltpu.VMEM((CH,), jnp.int32),
        pltpu.VMEM((CH,), jnp.int32),
        pltpu.VMEM((CH,), jnp.int32),
        pltpu.VMEM((CH, C), jnp.float32),
        pltpu.VMEM((CH, C), jnp.float32),
        pltpu.VMEM((CH, L), jnp.float32),
        pltpu.VMEM((CH,), jnp.float32),
        pltpu.VMEM((C,), jnp.float32),
        pltpu.VMEM((ZCH, C), jnp.float32),
        pltpu.VMEM_SHARED((NT, C), jnp.float32),
        pltpu.SemaphoreType.DMA,
        pltpu.SemaphoreType.DMA,
    ],
    compiler_params=_SC_PARAMS,
)
def _sc_layer2(xl_hbm, xr_hbm, src_hbm, dst_hbm, att_hbm, outn_hbm,
               srcv, dstv, gl, gr, si, bufL, bufR, accb, exb, attb,
               zb, tableN, semA, semB):
    c = lax.axis_index("c")
    s = lax.axis_index("s")
    iota = lax.iota(jnp.int32, L)
    zeros = jnp.zeros((L,), jnp.float32)
    lo = c * NPH

    pltpu.sync_copy(att_hbm, attb)
    attr = [attb[pl.ds(L * v, L)] for v in range(C // L)]

    @pl.loop(0, ZCH)
    def _zrow(r):
        for k in range(C // L):
            zb[r, pl.ds(L * k, L)] = zeros

    @pl.loop(0, RPT // 8)
    def _ztab(k):
        pltpu.sync_copy(zb.at[pl.ds(0, 8)],
                        tableN.at[pl.ds(s * RPT + k * 8, 8)])

    @pl.when(s < 2)
    def _ztabdum():
        pltpu.sync_copy(zb.at[pl.ds(0, 8)],
                        tableN.at[pl.ds(NPH + s * 8, 8)])
    plsc.subcore_barrier()

    @pl.loop(0, CPS)
    def _chunk(ci):
        base = s * (CPS * CH) + ci * CH
        h1 = pltpu.async_copy(src_hbm.at[pl.ds(base, CH)], srcv, semA)
        h2 = pltpu.async_copy(dst_hbm.at[pl.ds(base, CH)], dstv, semB)
        h1.wait()
        h2.wait()
        for k in range(CH // L):
            dl = dstv[pl.ds(L * k, L)]
            loc = dl - lo
            loc = jnp.where((loc >= 0) & (loc < NPH), loc, NPH + (dl & 15))
            gl[pl.ds(L * k, L)] = srcv[pl.ds(L * k, L)]
            gr[pl.ds(L * k, L)] = dl
            si[pl.ds(L * k, L)] = loc
        g1 = pltpu.async_copy(xl_hbm.at[gl], bufL, semA)
        g2 = pltpu.async_copy(xr_hbm.at[gr], bufR, semB)
        g1.wait()
        g2.wait()

        @pl.loop(0, CH)
        def _edge(j):
            acc = zeros
            for v in range(C // L):
                z = bufL[j, pl.ds(L * v, L)] + bufR[j, pl.ds(L * v, L)]
                lz = jnp.maximum(z, 0.2 * z)
                acc = acc + attr[v] * lz
            accb[j, :] = acc

        for g in range(CH // L):
            rows = g * L + iota
            tot = zeros
            for k in range(L):
                tot = tot + plsc.load_gather(
                    accb, [rows, jnp.full((L,), k, jnp.int32)])
            exb[pl.ds(g * L, L)] = jnp.exp(tot)

        @pl.loop(0, CH)
        def _scale(j):
            exs = plsc.load_gather(exb, [jnp.full((L,), j, jnp.int32)])
            for v in range(NC2 // L):
                bufL[j, pl.ds(L * v, L)] = bufL[j, pl.ds(L * v, L)] * exs
            bufL[j, pl.ds(NC2, L)] = jnp.where(iota == 0, exs, 0.0)

        pltpu.sync_copy(bufL, tableN.at[si], add=True)

    plsc.subcore_barrier()
    pltpu.sync_copy(tableN.at[pl.ds(s * RPT, RPT)],
                    outn_hbm.at[pl.ds(lo + s * RPT, RPT)])


def _fin_body(p2n_ref, b2_ref, o_ref):
    den = p2n_ref[:, NC2:NC2 + 1] + 1e-16
    o_ref[...] = p2n_ref[:, :NC] / den + b2_ref[...][None, :]


def _fin(p2n, bias2):
    return pl.pallas_call(
        _fin_body,
        grid=(NB,),
        in_specs=[
            pl.BlockSpec((ROWB, C), lambda i: (i, 0)),
            pl.BlockSpec((NC,), lambda i: (0,)),
        ],
        out_specs=pl.BlockSpec((ROWB, NC), lambda i: (i, 0)),
        out_shape=jax.ShapeDtypeStruct((NP, NC), jnp.float32),
    )(p2n, bias2)


def kernel(x, edge_index, W_l1, b_l1, W_r1, b_r1, att1, bias1,
           W_l2, b_l2, W_r2, b_r2, att2, bias2):
    xp = jnp.pad(x, ((0, NP - N), (0, 0)))
    loop = jnp.arange(N, dtype=jnp.int32)
    pad = jnp.full((EP - E - N,), N, jnp.int32)
    src = jnp.concatenate([edge_index[0], loop, pad])
    dst = jnp.concatenate([edge_index[1], loop, pad])

    xl1, xr1 = _proj1(xp, W_l1, b_l1, W_r1, b_r1)
    out1n, out1d = _sc_layer1(xl1, xr1, src, dst, att1)

    W_l2p = jnp.pad(W_l2, ((0, 0), (0, C - NC)))
    W_r2p = jnp.pad(W_r2, ((0, 0), (0, C - NC)))
    b_l2p = jnp.pad(b_l2, (0, C - NC))
    b_r2p = jnp.pad(b_r2, (0, C - NC))
    att2p = jnp.pad(att2[0], (0, C - NC))

    xl2, xr2 = _mid(out1n, out1d, W_l2p, b_l2p, W_r2p, b_r2p, bias1)
    p2n = _sc_layer2(xl2, xr2, src, dst, att2p)
    return _fin(p2n, bias2)[:N]
